# baseline (speedup 1.0000x reference)
import jax
import jax.numpy as jnp
from jax import lax
from jax.experimental import pallas as pl
from jax.experimental.pallas import tpu as pltpu
from jax.experimental.pallas import tpu_sc as plsc

B = 16384
NUM_COLS = 39
N_NUM = 13
N_CAT = 26
D = 32
PAD = 3
OUT_COLS = PAD + N_NUM + N_CAT * D
CAT_STRIDE = 100000

NC, NS = 2, 16
NW = NC * NS
RW = B // NW
CHUNK = 64
NCHUNK = RW // CHUNK
IDX_PER_CHUNK = CHUNK * N_CAT
NUM_PER_CHUNK = CHUNK * N_NUM
XW_PER_CHUNK = CHUNK * NUM_COLS


def _body(x_ref, table_ref, out_ref, xbuf, idxbuf, numbuf, gstage, sem):
    wid = lax.axis_index("s") * NC + lax.axis_index("c")
    lanes = lax.iota(jnp.int32, 16)
    lanes_x = lanes * NUM_COLS

    def chunk_body(k, carry):
        base = wid * RW + k * CHUNK
        pltpu.sync_copy(x_ref.at[pl.ds(base * NUM_COLS, XW_PER_CHUNK)], xbuf)

        for j in range(N_CAT):
            for rb in range(CHUNK // 16):
                v = plsc.load_gather(
                    xbuf, [lanes_x + (rb * 16 * NUM_COLS + N_NUM + j)])
                idxbuf[pl.ds(j * CHUNK + rb * 16, 16)] = (
                    v.astype(jnp.int32) + j * CAT_STRIDE)

        w0 = lanes >= N_NUM
        r = jnp.where(w0, jnp.ones((16,), jnp.int32),
                      jnp.zeros((16,), jnp.int32))
        c = jnp.where(w0, lanes - N_NUM, lanes)
        for _ in range(NUM_PER_CHUNK // 16):
            v = plsc.load_gather(xbuf, [r * NUM_COLS + c])
            plsc.store_scatter(numbuf, [r, c + PAD], v)
            t1 = c + (16 - N_NUM)
            w = t1 >= N_NUM
            r = jnp.where(w, r + 2, r + 1)
            c = jnp.where(w, t1 - N_NUM, t1)

        copies = [
            pltpu.async_copy(
                table_ref.at[idxbuf.at[pl.ds(j * CHUNK, CHUNK)]],
                gstage.at[pl.ds(j * CHUNK, CHUNK), :],
                sem,
            )
            for j in range(N_CAT)
        ]
        for cp in copies:
            cp.wait()

        for j in range(N_CAT):
            pltpu.sync_copy(
                gstage.at[pl.ds(j * CHUNK, CHUNK), :],
                out_ref.at[pl.ds(base, CHUNK),
                           pl.ds(PAD + N_NUM + j * D, D)])
        pltpu.sync_copy(numbuf,
                        out_ref.at[pl.ds(base, CHUNK), pl.ds(0, PAD + N_NUM)])
        return carry

    lax.fori_loop(0, NCHUNK, chunk_body, 0)


@jax.jit
def kernel(x, table):
    run = pl.kernel(
        _body,
        out_type=jax.ShapeDtypeStruct((B, OUT_COLS), jnp.float32),
        mesh=plsc.VectorSubcoreMesh(core_axis_name="c", subcore_axis_name="s"),
        compiler_params=pltpu.CompilerParams(use_tc_tiling_on_sc=False,
                                             needs_layout_passes=False),
        scratch_types=[
            pltpu.VMEM((XW_PER_CHUNK,), jnp.float32),
            pltpu.VMEM((IDX_PER_CHUNK,), jnp.int32),
            pltpu.VMEM((CHUNK, PAD + N_NUM), jnp.float32),
            pltpu.VMEM((IDX_PER_CHUNK, D), jnp.float32),
            pltpu.SemaphoreType.DMA,
        ],
    )
    out = run(x.reshape(-1), table)
    return out[:, PAD:]
